# baseline (speedup 1.0000x reference)
import functools

import jax
import jax.numpy as jnp
from jax import lax
from jax.experimental import pallas as pl
from jax.experimental.pallas import tpu as pltpu
from jax.experimental.pallas import tpu_sc as plsc

VOCAB = 1000000
EMB_DIM = 32
BATCH = 4096
SEQ = 50
TOTAL = BATCH * SEQ

_info = plsc.get_sparse_core_info()
NC, NS = _info.num_cores, _info.num_subcores
NW = NC * NS
B_PER_W = TOTAL // NW
CHUNK = 3200
N_CHUNKS = B_PER_W // CHUNK


def _emb_kernel(idx_hbm, table_hbm, out_hbm, idx_v, rows_v, sem):
    wid = lax.axis_index("s") * NC + lax.axis_index("c")
    base = wid * B_PER_W

    def body(g, _):
        off = base + g * CHUNK
        pltpu.sync_copy(idx_hbm.at[pl.ds(off, CHUNK)], idx_v)
        pltpu.async_copy(table_hbm.at[idx_v], rows_v, sem).wait()
        pltpu.sync_copy(rows_v, out_hbm.at[pl.ds(off, CHUNK)])
        return ()

    lax.fori_loop(0, N_CHUNKS, body, ())


@jax.jit
def _emb(word_ids, table):
    idx = word_ids.reshape(TOTAL)
    mesh = plsc.VectorSubcoreMesh(core_axis_name="c", subcore_axis_name="s")
    k = functools.partial(
        pl.kernel,
        mesh=mesh,
        out_type=jax.ShapeDtypeStruct((TOTAL, EMB_DIM), jnp.float32),
        scratch_types=[
            pltpu.VMEM((CHUNK,), jnp.int32),
            pltpu.VMEM((CHUNK, EMB_DIM), jnp.float32),
            pltpu.SemaphoreType.DMA,
        ],
        compiler_params=pltpu.CompilerParams(use_tc_tiling_on_sc=False),
    )(_emb_kernel)
    out = k(idx, table)
    return out.reshape(BATCH, SEQ, EMB_DIM)


def kernel(word_ids, table):
    return _emb(word_ids, table)


# speedup vs baseline: 1.0008x; 1.0008x over previous
import functools

import jax
import jax.numpy as jnp
from jax import lax
from jax.experimental import pallas as pl
from jax.experimental.pallas import tpu as pltpu
from jax.experimental.pallas import tpu_sc as plsc

VOCAB = 1000000
EMB_DIM = 32
BATCH = 4096
SEQ = 50
TOTAL = BATCH * SEQ

_info = plsc.get_sparse_core_info()
NC, NS = _info.num_cores, _info.num_subcores
NW = NC * NS
B_PER_W = TOTAL // NW
CHUNK = 1600
N_CHUNKS = B_PER_W // CHUNK


def _emb_kernel(idx_hbm, table_hbm, out_hbm,
                idx0, idx1, rows0, rows1, gsem, wsem):
    wid = lax.axis_index("s") * NC + lax.axis_index("c")
    base = wid * B_PER_W
    idx_v = [idx0, idx1]
    rows_v = [rows0, rows1]

    pltpu.sync_copy(idx_hbm.at[pl.ds(base, CHUNK)], idx_v[0])
    gathers = [pltpu.async_copy(table_hbm.at[idx_v[0]], rows_v[0], gsem)]
    writes = []
    for g in range(N_CHUNKS):
        if g + 1 < N_CHUNKS:
            off = base + (g + 1) * CHUNK
            b = (g + 1) % 2
            pltpu.sync_copy(idx_hbm.at[pl.ds(off, CHUNK)], idx_v[b])
            if g >= 1:
                writes[g - 1].wait()
            gathers.append(
                pltpu.async_copy(table_hbm.at[idx_v[b]], rows_v[b], gsem))
        gathers[g].wait()
        writes.append(
            pltpu.async_copy(rows_v[g % 2],
                             out_hbm.at[pl.ds(base + g * CHUNK, CHUNK)],
                             wsem))
    writes[N_CHUNKS - 2].wait()
    writes[N_CHUNKS - 1].wait()


@jax.jit
def _emb(word_ids, table):
    idx = word_ids.reshape(TOTAL)
    mesh = plsc.VectorSubcoreMesh(core_axis_name="c", subcore_axis_name="s")
    k = functools.partial(
        pl.kernel,
        mesh=mesh,
        out_type=jax.ShapeDtypeStruct((TOTAL, EMB_DIM), jnp.float32),
        scratch_types=[
            pltpu.VMEM((CHUNK,), jnp.int32),
            pltpu.VMEM((CHUNK,), jnp.int32),
            pltpu.VMEM((CHUNK, EMB_DIM), jnp.float32),
            pltpu.VMEM((CHUNK, EMB_DIM), jnp.float32),
            pltpu.SemaphoreType.DMA,
            pltpu.SemaphoreType.DMA,
        ],
        compiler_params=pltpu.CompilerParams(use_tc_tiling_on_sc=False),
    )(_emb_kernel)
    out = k(idx, table)
    return out.reshape(BATCH, SEQ, EMB_DIM)


def kernel(word_ids, table):
    return _emb(word_ids, table)


# speedup vs baseline: 1.2260x; 1.2250x over previous
import functools

import jax
import jax.numpy as jnp
from jax import lax
from jax.experimental import pallas as pl
from jax.experimental.pallas import tpu as pltpu
from jax.experimental.pallas import tpu_sc as plsc

VOCAB = 1000000
EMB_DIM = 32
BATCH = 4096
SEQ = 50

_info = plsc.get_sparse_core_info()
NC, NS = _info.num_cores, _info.num_subcores
NW = NC * NS
ROWS_PER_W = BATCH // NW
ROWS_PER_CHUNK = 32
CHUNK = ROWS_PER_CHUNK * SEQ
N_CHUNKS = ROWS_PER_W // ROWS_PER_CHUNK


def _emb_kernel(idx_hbm, table_hbm, out_hbm,
                idx0, idx1, rows0, rows1, gsem, wsem):
    wid = lax.axis_index("s") * NC + lax.axis_index("c")
    row_base = wid * ROWS_PER_W
    idx_v = [idx0, idx1]
    rows_v = [rows0, rows1]

    def stage_idx(g, b):
        r0 = row_base + g * ROWS_PER_CHUNK
        pltpu.sync_copy(idx_hbm.at[pl.ds(r0, ROWS_PER_CHUNK), :], idx_v[b])

    def fire_gather(g, b):
        return [
            pltpu.async_copy(
                table_hbm.at[idx_v[b].at[r]],
                rows_v[b].at[r],
                gsem,
            )
            for r in range(ROWS_PER_CHUNK)
        ]

    def fire_write(g, b):
        r0 = row_base + g * ROWS_PER_CHUNK
        return pltpu.async_copy(
            rows_v[b],
            out_hbm.at[pl.ds(r0, ROWS_PER_CHUNK), :, :],
            wsem)

    stage_idx(0, 0)
    gathers = [fire_gather(0, 0)]
    writes = []
    for g in range(N_CHUNKS):
        if g + 1 < N_CHUNKS:
            b = (g + 1) % 2
            stage_idx(g + 1, b)
            if g >= 1:
                writes[g - 1].wait()
            gathers.append(fire_gather(g + 1, b))
        for d in gathers[g]:
            d.wait()
        writes.append(fire_write(g, g % 2))
    writes[N_CHUNKS - 2].wait()
    writes[N_CHUNKS - 1].wait()


@jax.jit
def _emb(word_ids, table):
    mesh = plsc.VectorSubcoreMesh(core_axis_name="c", subcore_axis_name="s")
    k = functools.partial(
        pl.kernel,
        mesh=mesh,
        out_type=jax.ShapeDtypeStruct((BATCH, SEQ, EMB_DIM), jnp.float32),
        scratch_types=[
            pltpu.VMEM((ROWS_PER_CHUNK, SEQ), jnp.int32),
            pltpu.VMEM((ROWS_PER_CHUNK, SEQ), jnp.int32),
            pltpu.VMEM((ROWS_PER_CHUNK, SEQ, EMB_DIM), jnp.float32),
            pltpu.VMEM((ROWS_PER_CHUNK, SEQ, EMB_DIM), jnp.float32),
            pltpu.SemaphoreType.DMA,
            pltpu.SemaphoreType.DMA,
        ],
        compiler_params=pltpu.CompilerParams(use_tc_tiling_on_sc=False),
    )(_emb_kernel)
    return k(word_ids, table)


def kernel(word_ids, table):
    return _emb(word_ids, table)
